# baseline (speedup 1.0000x reference)
import functools

import jax
import jax.numpy as jnp
from jax import lax
from jax.experimental import pallas as pl
from jax.experimental.pallas import tpu as pltpu, tpu_sc as plsc

_G = 8
_CH = 512


def _body(feat_ref, hid_ref, w4_ref, wa4_ref, bf_ref, wh_ref, bh_ref,
          ba_ref, wb_ref, bb_ref, alpha_ref, lp_ref, fidx_ref, beta_ref):
    G, L, D = feat_ref.shape
    A = bf_ref.shape[1]
    npk = w4_ref.shape[0]
    rows_grp = npk * _CH
    g_grp = rows_grp // L
    ng = G * L // rows_grp
    X = feat_ref[...].reshape(G * L, D)
    hh = hid_ref[...]
    BQ = jnp.dot(hh, wh_ref[...]) + bh_ref[...] + bf_ref[...]
    Beta = jax.nn.sigmoid(jnp.dot(hh, wb_ref[...]) + bb_ref[...])
    parts = []
    for j in range(ng):
        g0 = j * g_grp
        base = j * rows_grp
        U = jnp.dot(X[base:base + _CH], w4_ref[0])
        for k in range(1, npk):
            U = U + jnp.dot(X[base + k * _CH:base + (k + 1) * _CH],
                            w4_ref[k])
        br = jnp.concatenate(
            [BQ[g0 + (k * _CH) // L:g0 + (k * _CH) // L + 1]
             for k in range(npk)], axis=1)
        T = jnp.tanh(U + br)
        parts.append(jnp.transpose(jnp.dot(T, wa4_ref[...])))
    E2 = (jnp.concatenate(parts, axis=0).reshape(G, L)
          + ba_ref[...])
    m = jnp.max(E2, axis=1, keepdims=True)
    p = jnp.exp(E2 - m)
    s = jnp.sum(p, axis=1, keepdims=True)
    alpha2 = p / s
    amax = jnp.max(alpha2, axis=1, keepdims=True)
    iota2 = lax.broadcasted_iota(jnp.int32, E2.shape, 1)
    idxc = jnp.min(jnp.where(alpha2 == amax, iota2, L),
                   axis=1, keepdims=True)
    giota = lax.broadcasted_iota(jnp.int32, (G, 1), 0)
    fidx_ref[...] = idxc + (pl.program_id(0) * G + giota) * L
    alpha_ref[...] = alpha2
    lp_ref[...] = jnp.log(amax)
    beta_ref[...] = jnp.broadcast_to(Beta, (G, beta_ref.shape[1]))


def _tc_scores(features, hidden, Wf, bf, Wh, bh, Wa, ba, Wb, bb,
               b0=0, nb=None):
    B, L, D = features.shape
    nb = B if nb is None else nb
    H = hidden.shape[1]
    A = Wf.shape[1]
    f32 = jnp.float32
    G = _G
    s0 = b0 // G
    NP = 128 // A
    W4 = jnp.stack([jnp.pad(Wf, ((0, 0), (A * k, 128 - A * (k + 1))))
                    for k in range(NP)])
    Wa4 = sum(jnp.pad(Wa, ((A * k, 128 - A * (k + 1)),
                           (k, NP - k - 1))) for k in range(NP))
    return pl.pallas_call(
        _body,
        grid=(nb // G,),
        in_specs=[
            pl.BlockSpec((G, L, D), lambda b: (b + s0, 0, 0)),
            pl.BlockSpec((G, H), lambda b: (b + s0, 0)),
            pl.BlockSpec((NP, D, 128), lambda b: (0, 0, 0)),
            pl.BlockSpec((128, NP), lambda b: (0, 0)),
            pl.BlockSpec((1, A), lambda b: (0, 0)),
            pl.BlockSpec((H, A), lambda b: (0, 0)),
            pl.BlockSpec((1, A), lambda b: (0, 0)),
            pl.BlockSpec((1, 1), lambda b: (0, 0)),
            pl.BlockSpec((H, 1), lambda b: (0, 0)),
            pl.BlockSpec((1, 1), lambda b: (0, 0)),
        ],
        out_specs=[
            pl.BlockSpec((G, L), lambda b: (b, 0)),
            pl.BlockSpec((G, 1), lambda b: (b, 0)),
            pl.BlockSpec((G, 1), lambda b: (b, 0)),
            pl.BlockSpec((G, 16), lambda b: (b, 0)),
        ],
        out_shape=[
            jax.ShapeDtypeStruct((nb, L), f32),
            jax.ShapeDtypeStruct((nb, 1), f32),
            jax.ShapeDtypeStruct((nb, 1), jnp.int32),
            jax.ShapeDtypeStruct((nb, 16), f32),
        ],
    )(features, hidden, W4, Wa4, bf.reshape(1, A), Wh,
      bh.reshape(1, A), ba.reshape(1, 1), Wb, bb.reshape(1, 1))


def _sc_gather(feat2d, fidx, beta16):
    R, D = feat2d.shape
    B = fidx.shape[0]
    info = plsc.get_sparse_core_info()
    NC, NS, LN = info.num_cores, info.num_subcores, info.num_lanes
    RPW = 8
    nw_active = B // RPW
    mesh = plsc.VectorSubcoreMesh(core_axis_name="c", subcore_axis_name="s")

    @functools.partial(
        pl.kernel, mesh=mesh,
        out_type=jax.ShapeDtypeStruct((B, D), jnp.float32),
        scratch_types=[
            pltpu.VMEM((RPW,), jnp.int32),
            pltpu.VMEM((RPW, 16), jnp.float32),
            pltpu.VMEM((RPW, D), jnp.float32),
            pltpu.SemaphoreType.DMA,
        ],
    )
    def k(feat_hbm, idx_hbm, beta_hbm, out_hbm, idx_v, beta_v, rows_v, sem):
        wid = lax.axis_index("s") * NC + lax.axis_index("c")

        @pl.when(wid < nw_active)
        def _():
            base = wid * RPW
            pltpu.sync_copy(idx_hbm.at[pl.ds(base, RPW)], idx_v)
            pltpu.sync_copy(beta_hbm.at[pl.ds(base, RPW)], beta_v)
            pltpu.async_copy(feat_hbm.at[idx_v], rows_v, sem).wait()
            for r in range(RPW):
                bv = beta_v[r, :]
                for c in range(0, D, LN):
                    rows_v[r, pl.ds(c, LN)] = rows_v[r, pl.ds(c, LN)] * bv
            pltpu.sync_copy(rows_v, out_hbm.at[pl.ds(base, RPW)])

    return k(feat2d, fidx, beta16)


def kernel(features, hidden, Wf, bf, Wh, bh, Wa, ba, Wb, bb):
    B, L, D = features.shape
    feat2d = features.reshape(B * L, D)
    hb = B // 2
    outs = []
    for h in range(2):
        lo = h * hb
        alpha_h, lp_h, fidx_h, beta_h = _tc_scores(
            features, hidden, Wf, bf, Wh, bh, Wa, ba, Wb, bb,
            b0=lo, nb=hb)
        ctx_h = _sc_gather(feat2d, fidx_h.reshape(hb) + lo * L, beta_h)
        outs.append((ctx_h, alpha_h, lp_h))
    ctx = jnp.concatenate([o[0] for o in outs])
    alpha = jnp.concatenate([o[1] for o in outs])
    lp = jnp.concatenate([o[2] for o in outs])
    return ctx, alpha, lp.reshape(B)


# speedup vs baseline: 1.1610x; 1.1610x over previous
import functools

import jax
import jax.numpy as jnp
from jax import lax
from jax.experimental import pallas as pl
from jax.experimental.pallas import tpu as pltpu, tpu_sc as plsc

_G = 8
_CH = 512


def _body(feat_ref, hid_ref, w4_ref, wa4_ref, bf_ref, wh_ref, bh_ref,
          ba_ref, wb_ref, bb_ref, alpha_ref, lp_ref, fidx_ref, beta_ref):
    G, L, D = feat_ref.shape
    A = bf_ref.shape[1]
    npk = w4_ref.shape[0]
    rows_grp = npk * _CH
    g_grp = rows_grp // L
    ng = G * L // rows_grp
    X = feat_ref[...].reshape(G * L, D)
    hh = hid_ref[...]
    BQ = jnp.dot(hh, wh_ref[...]) + bh_ref[...] + bf_ref[...]
    Beta = jax.nn.sigmoid(jnp.dot(hh, wb_ref[...]) + bb_ref[...])
    parts = []
    for j in range(ng):
        g0 = j * g_grp
        base = j * rows_grp
        U = jnp.dot(X[base:base + _CH], w4_ref[0])
        for k in range(1, npk):
            U = U + jnp.dot(X[base + k * _CH:base + (k + 1) * _CH],
                            w4_ref[k])
        br = jnp.concatenate(
            [BQ[g0 + (k * _CH) // L:g0 + (k * _CH) // L + 1]
             for k in range(npk)], axis=1)
        T = jnp.tanh(U + br)
        parts.append(jnp.transpose(jnp.dot(T, wa4_ref[...])))
    E2 = (jnp.concatenate(parts, axis=0).reshape(G, L)
          + ba_ref[...])
    m = jnp.max(E2, axis=1, keepdims=True)
    p = jnp.exp(E2 - m)
    s = jnp.sum(p, axis=1, keepdims=True)
    alpha2 = p / s
    amax = jnp.max(alpha2, axis=1, keepdims=True)
    iota2 = lax.broadcasted_iota(jnp.int32, E2.shape, 1)
    idxc = jnp.min(jnp.where(alpha2 == amax, iota2, L),
                   axis=1, keepdims=True)
    giota = lax.broadcasted_iota(jnp.int32, (G, 1), 0)
    fidx_ref[...] = idxc + (pl.program_id(0) * G + giota) * L
    alpha_ref[...] = alpha2
    lp_ref[...] = jnp.log(amax)
    beta_ref[...] = jnp.broadcast_to(Beta, (G, beta_ref.shape[1]))


def _tc_scores(features, hidden, Wf, bf, Wh, bh, Wa, ba, Wb, bb):
    B, L, D = features.shape
    H = hidden.shape[1]
    A = Wf.shape[1]
    f32 = jnp.float32
    G = _G
    NP = 128 // A
    W4 = jnp.stack([jnp.pad(Wf, ((0, 0), (A * k, 128 - A * (k + 1))))
                    for k in range(NP)])
    Wa4 = sum(jnp.pad(Wa, ((A * k, 128 - A * (k + 1)),
                           (k, NP - k - 1))) for k in range(NP))
    return pl.pallas_call(
        _body,
        grid=(B // G,),
        in_specs=[
            pl.BlockSpec((G, L, D), lambda b: (b, 0, 0)),
            pl.BlockSpec((G, H), lambda b: (b, 0)),
            pl.BlockSpec((NP, D, 128), lambda b: (0, 0, 0)),
            pl.BlockSpec((128, NP), lambda b: (0, 0)),
            pl.BlockSpec((1, A), lambda b: (0, 0)),
            pl.BlockSpec((H, A), lambda b: (0, 0)),
            pl.BlockSpec((1, A), lambda b: (0, 0)),
            pl.BlockSpec((1, 1), lambda b: (0, 0)),
            pl.BlockSpec((H, 1), lambda b: (0, 0)),
            pl.BlockSpec((1, 1), lambda b: (0, 0)),
        ],
        out_specs=[
            pl.BlockSpec((G, L), lambda b: (b, 0)),
            pl.BlockSpec((G, 1), lambda b: (b, 0)),
            pl.BlockSpec((G, 1), lambda b: (b, 0)),
            pl.BlockSpec((G, 16), lambda b: (b, 0)),
        ],
        out_shape=[
            jax.ShapeDtypeStruct((B, L), f32),
            jax.ShapeDtypeStruct((B, 1), f32),
            jax.ShapeDtypeStruct((B, 1), jnp.int32),
            jax.ShapeDtypeStruct((B, 16), f32),
        ],
    )(features, hidden, W4, Wa4, bf.reshape(1, A), Wh,
      bh.reshape(1, A), ba.reshape(1, 1), Wb, bb.reshape(1, 1))


def _sc_gather(feat2d, fidx, beta16):
    R, D = feat2d.shape
    B = fidx.shape[0]
    info = plsc.get_sparse_core_info()
    NC, NS, LN = info.num_cores, info.num_subcores, info.num_lanes
    RPW = 8
    nw_active = B // RPW
    mesh = plsc.VectorSubcoreMesh(core_axis_name="c", subcore_axis_name="s")

    @functools.partial(
        pl.kernel, mesh=mesh,
        out_type=jax.ShapeDtypeStruct((B, D), jnp.float32),
        scratch_types=[
            pltpu.VMEM((RPW,), jnp.int32),
            pltpu.VMEM((RPW, 16), jnp.float32),
            pltpu.VMEM((RPW, D), jnp.float32),
            pltpu.SemaphoreType.DMA,
        ],
    )
    def k(feat_hbm, idx_hbm, beta_hbm, out_hbm, idx_v, beta_v, rows_v, sem):
        wid = lax.axis_index("s") * NC + lax.axis_index("c")

        @pl.when(wid < nw_active)
        def _():
            base = wid * RPW
            pltpu.sync_copy(idx_hbm.at[pl.ds(base, RPW)], idx_v)
            pltpu.sync_copy(beta_hbm.at[pl.ds(base, RPW)], beta_v)
            pltpu.async_copy(feat_hbm.at[idx_v], rows_v, sem).wait()
            for r in range(RPW):
                bv = beta_v[r, :]
                for c in range(0, D, LN):
                    rows_v[r, pl.ds(c, LN)] = rows_v[r, pl.ds(c, LN)] * bv
            pltpu.sync_copy(rows_v, out_hbm.at[pl.ds(base, RPW)])

    return k(feat2d, fidx, beta16)


def kernel(features, hidden, Wf, bf, Wh, bh, Wa, ba, Wb, bb):
    B, L, D = features.shape
    alpha, lp, fidx, beta = _tc_scores(features, hidden, Wf, bf, Wh, bh,
                                       Wa, ba, Wb, bb)
    ctx = _sc_gather(features.reshape(B * L, D), fidx.reshape(B), beta)
    return ctx, alpha, lp.reshape(B)
